# Initial kernel scaffold; baseline (speedup 1.0000x reference)
import jax
import jax.numpy as jnp
from jax.experimental import pallas as pl


def kernel(anchors, gt_bbox):
    raise NotImplementedError("write your pallas kernel here")


# speedup vs baseline: 3.5631x; 3.5631x over previous
import jax
import jax.numpy as jnp
from jax import lax
from jax.experimental import pallas as pl

N0 = 20000
R, C = 160, 128
NP = R * C
G = 64
IMG_W = 1024.0
IMG_H = 1024.0
NEG_THRESH = 0.3
POS_THRESH = 0.7
N_FG = 128
N_BG = 128


def _body(a_ref, g_ref, dx_ref, dy_ref, dw_ref, dh_ref, lab_ref):
    ax1 = a_ref[0:1, :]
    ay1 = a_ref[1:2, :]
    ax2 = a_ref[2:3, :]
    ay2 = a_ref[3:4, :]
    gx1 = g_ref[:, 0:1]
    gy1 = g_ref[:, 1:2]
    gx2 = g_ref[:, 2:3]
    gy2 = g_ref[:, 3:4]

    inside = (ax1 >= 0.0) & (ay1 >= 0.0) & (ax2 <= IMG_W) & (ay2 <= IMG_H)

    iw = jnp.maximum(jnp.minimum(ax2, gx2) - jnp.maximum(ax1, gx1) + 1.0, 0.0)
    ih = jnp.maximum(jnp.minimum(ay2, gy2) - jnp.maximum(ay1, gy1) + 1.0, 0.0)
    inter = iw * ih
    area_a = (ax2 - ax1 + 1.0) * (ay2 - ay1 + 1.0)
    area_g = (gx2 - gx1 + 1.0) * (gy2 - gy1 + 1.0)
    union = area_a + area_g - inter
    iou = inter / union
    iou = jnp.where(inside, iou, -1.0)

    maxN = jnp.max(iou, axis=0, keepdims=True)
    gi = lax.broadcasted_iota(jnp.int32, (G, NP), 0)
    amax_a = jnp.min(jnp.where(iou == maxN, gi, G), axis=0, keepdims=True)

    colmax = jnp.max(iou, axis=1, keepdims=True)
    ai = lax.broadcasted_iota(jnp.int32, (G, NP), 1)
    amax_g = jnp.min(jnp.where(iou == colmax, ai, NP), axis=1, keepdims=True)

    member = jnp.max((ai == amax_g).astype(jnp.int32), axis=0, keepdims=True) > 0

    valid = lax.broadcasted_iota(jnp.int32, (1, NP), 1) < N0
    labels0 = jnp.where(
        member | (maxN > POS_THRESH), 1.0,
        jnp.where(maxN < NEG_THRESH, 0.0, -1.0))

    pos = (labels0 == 1.0) & valid
    neg = (labels0 == 0.0) & valid

    posf = pos.astype(jnp.float32).reshape(R, C)
    negf = neg.astype(jnp.float32).reshape(R, C)

    ii = lax.broadcasted_iota(jnp.int32, (C, C), 0)
    jj = lax.broadcasted_iota(jnp.int32, (C, C), 1)
    upper = (ii <= jj).astype(jnp.float32)
    rr = lax.broadcasted_iota(jnp.int32, (R, R), 0)
    cc = lax.broadcasted_iota(jnp.int32, (R, R), 1)
    strict = (cc < rr).astype(jnp.float32)

    def cum2d(m):
        rowcs = jnp.dot(m, upper, preferred_element_type=jnp.float32)
        rowsum = jnp.broadcast_to(rowcs[:, C - 1:C], (R, C))
        off = jnp.dot(strict, rowsum, preferred_element_type=jnp.float32)
        return rowcs + off

    poscum = cum2d(posf)
    negcum = cum2d(negf)

    lab0 = labels0.reshape(R, C)
    lab1 = jnp.where((posf > 0.0) & (poscum > float(N_FG)), -1.0, lab0)
    lab2 = jnp.where((negf > 0.0) & (negcum > float(N_BG)), -1.0, lab1)
    insidef = inside.astype(jnp.float32)
    lab_ref[...] = jnp.where(insidef.reshape(R, C) > 0.0, lab2, -1.0)

    gw = gx2 - gx1 + 1.0
    gh = gy2 - gy1 + 1.0
    gcx = gx1 + 0.5 * gw
    gcy = gy1 + 0.5 * gh
    onehot = (gi == amax_a).astype(jnp.float32)
    m_gw = jnp.sum(onehot * gw, axis=0, keepdims=True)
    m_gh = jnp.sum(onehot * gh, axis=0, keepdims=True)
    m_gcx = jnp.sum(onehot * gcx, axis=0, keepdims=True)
    m_gcy = jnp.sum(onehot * gcy, axis=0, keepdims=True)

    ew = ax2 - ax1 + 1.0
    eh = ay2 - ay1 + 1.0
    ecx = ax1 + 0.5 * ew
    ecy = ay1 + 0.5 * eh
    dx = jnp.where(inside, (m_gcx - ecx) / ew, 0.0)
    dy = jnp.where(inside, (m_gcy - ecy) / eh, 0.0)
    dw = jnp.where(inside, jnp.log(m_gw / ew), 0.0)
    dh = jnp.where(inside, jnp.log(m_gh / eh), 0.0)
    dx_ref[...] = dx.reshape(R, C)
    dy_ref[...] = dy.reshape(R, C)
    dw_ref[...] = dw.reshape(R, C)
    dh_ref[...] = dh.reshape(R, C)


def _run(anchors, gt_bbox, interpret=False):
    pad = jnp.full((NP - N0, 4), 0.0, dtype=jnp.float32)
    pad = pad + jnp.array([-100.0, -100.0, -50.0, -50.0], dtype=jnp.float32)
    a_t = jnp.concatenate([anchors, pad], axis=0).T
    outs = pl.pallas_call(
        _body,
        out_shape=[jax.ShapeDtypeStruct((R, C), jnp.float32)] * 5,
        interpret=interpret,
    )(a_t, gt_bbox)
    dx, dy, dw, dh, lab = outs
    delta = jnp.stack(
        [dx.reshape(-1), dy.reshape(-1), dw.reshape(-1), dh.reshape(-1)],
        axis=1)[:N0]
    labels = lab.reshape(-1)[:N0]
    return delta, labels


def kernel(anchors, gt_bbox):
    return _run(anchors, gt_bbox)
